# Initial kernel scaffold; baseline (speedup 1.0000x reference)
import jax
import jax.numpy as jnp
from jax.experimental import pallas as pl


def kernel(x, W1, W2, S1w, S1b, S2w, S2b, S3w, S3b):
    raise NotImplementedError("write your pallas kernel here")


# speedup vs baseline: 3.4649x; 3.4649x over previous
import jax
import jax.numpy as jnp
from jax import lax
from jax.experimental import pallas as pl

_INT_MIN = -2147483648


def _skey(q):
    b = lax.bitcast_convert_type(q, jnp.int32)
    return jnp.where(b < 0, jnp.int32(_INT_MIN) - b, b)


def _kth_largest(keys, k, red):
    cnt0 = red(keys >= 0)
    p = jnp.where(cnt0 >= k, jnp.int32(0), jnp.int32(_INT_MIN))
    for beta in range(30, -1, -1):
        c = p + jnp.int32(1 << beta)
        cnt = red(keys >= c)
        p = jnp.where(cnt >= k, c, p)
    return p


def _cumsum_last(x):
    n = x.shape[-1]
    s = x
    sh = 1
    while sh < n:
        z = jnp.zeros(x.shape[:-1] + (sh,), x.dtype)
        s = s + jnp.concatenate([z, s[..., :n - sh]], axis=-1)
        sh *= 2
    return s


def _topk_mask2d(q, k):
    keys = _skey(q)
    red = lambda m: jnp.sum(m.astype(jnp.int32), axis=1, keepdims=True)
    t = _kth_largest(keys, k, red)
    gt = keys > t
    eq = keys == t
    extra = k - red(gt)
    rank = _cumsum_last(eq.astype(jnp.int32))
    return gt | (eq & (rank <= extra))


def _topk_mask3d(q, k):
    keys = _skey(q)

    def red(m):
        return jnp.sum(jnp.sum(m.astype(jnp.int32), axis=2, keepdims=True),
                       axis=1, keepdims=True)

    t = _kth_largest(keys, k, red)
    gt = keys > t
    eq = keys == t
    extra = k - red(gt)
    eqi = eq.astype(jnp.int32)
    within = _cumsum_last(eqi)
    rowtot = jnp.sum(eqi, axis=2)
    exc = _cumsum_last(rowtot) - rowtot
    rank = within + exc[:, :, None]
    return gt | (eq & (rank <= extra))


def _signed_absmax(pmax, nmin):
    return jnp.where(pmax >= -nmin, pmax, nmin)


def _body(x_ref, W1_ref, W2_ref, S1w_ref, S1b_ref, S2w_ref, S2b_ref,
          S3w_ref, S3b_ref, out_ref, sig_ref):
    f32 = jnp.float32
    x = x_ref[...]
    W1 = W1_ref[...]
    W2 = W2_ref[...]

    dot = lambda a, b: lax.dot_general(
        a, b, (((1,), (1,)), ((), ())), preferred_element_type=f32)

    h = jnp.maximum(dot(x, S1w_ref[...]) + S1b_ref[...], 0.0)
    h = jnp.maximum(dot(h, S2w_ref[...]) + S2b_ref[...], 0.0)
    sigma = jnp.tanh(dot(h, S3w_ref[...]) + S3b_ref[...])
    sig_ref[...] = sigma

    sw = jnp.mean(sigma[:, 0:5], axis=1, keepdims=True)
    pw = jnp.mean(sigma[:, 5:10], axis=1, keepdims=True)
    dw = jnp.mean(sigma[:, 10:15], axis=1, keepdims=True)
    tot = jnp.abs(sw) + jnp.abs(pw) + jnp.abs(dw) + 1e-8
    sw, pw, dw = sw / tot, pw / tot, dw / tot

    absW1 = jnp.abs(W1)
    c1 = jnp.sqrt(jnp.sum(W1 * W1, axis=0, keepdims=True))
    mu1 = jnp.mean(W1, axis=0, keepdims=True)
    v1 = jnp.sum((W1 - mu1) ** 2, axis=0, keepdims=True) * (1.0 / 63.0)
    absx = jnp.abs(x)
    sp1 = jnp.sum((absW1[None, :, :] * absx[:, None, :] < 0.1).astype(f32),
                  axis=1) * (1.0 / 64.0)
    q1 = sw * (absx * c1) + pw * sp1 + dw * (absx * absx * v1)
    mask1 = _topk_mask2d(q1, 100)

    dt = jnp.tanh(x[:, None, :] * W1[None, :, :])
    absd = jnp.abs(dt)

    absW2 = jnp.abs(W2)
    c2 = jnp.sqrt(jnp.sum(W2 * W2, axis=0, keepdims=True))
    mu2 = jnp.mean(W2, axis=0, keepdims=True)
    v2 = jnp.sum((W2 - mu2) ** 2, axis=0, keepdims=True) * (1.0 / 31.0)
    cnt = jnp.zeros_like(dt)
    for j2 in range(32):
        cnt = cnt + (absW2[j2:j2 + 1, :, None] * absd < 0.1).astype(f32)
    q2 = (sw[:, :, None] * (absd * c2[:, :, None])
          + pw[:, :, None] * (cnt * (1.0 / 32.0))
          + dw[:, :, None] * (absd * absd * v2[:, :, None]))
    q2 = jnp.where(mask1[:, None, :], q2, -jnp.inf)
    mask2 = _topk_mask3d(q2, 100)

    dm = jnp.where(mask2, dt, 0.0)
    m1 = jnp.sum(dm, axis=2)
    mean_out = dot(m1, W2) * (1.0 / 100.0)
    pmax = jnp.max(dm, axis=2)
    nmin = jnp.min(dm, axis=2)
    dstar = _signed_absmax(pmax, nmin)
    a = W2[None, :, :] * dstar[:, None, :]
    maxval = _signed_absmax(jnp.max(a, axis=2), jnp.min(a, axis=2))
    alpha = (sigma[:, 15:16] + 1.0) * 0.5
    out_ref[...] = (1.0 - alpha) * mean_out + alpha * maxval


def kernel(x, W1, W2, S1w, S1b, S2w, S2b, S3w, S3b):
    B = x.shape[0]
    Bb = 64
    bcast = lambda shape: pl.BlockSpec(shape, lambda i: (0,) * len(shape))
    out_shape = (jax.ShapeDtypeStruct((B, 32), jnp.float32),
                 jax.ShapeDtypeStruct((B, 16), jnp.float32))
    return pl.pallas_call(
        _body,
        grid=(B // Bb,),
        in_specs=[
            pl.BlockSpec((Bb, 128), lambda i: (i, 0)),
            bcast((64, 128)),
            bcast((32, 64)),
            bcast((64, 128)),
            bcast((1, 64)),
            bcast((32, 64)),
            bcast((1, 32)),
            bcast((16, 32)),
            bcast((1, 16)),
        ],
        out_specs=(pl.BlockSpec((Bb, 32), lambda i: (i, 0)),
                   pl.BlockSpec((Bb, 16), lambda i: (i, 0))),
        out_shape=out_shape,
    )(x, W1, W2, S1w, S1b.reshape(1, 64), S2w, S2b.reshape(1, 32),
      S3w, S3b.reshape(1, 16))


# speedup vs baseline: 3.9024x; 1.1263x over previous
import jax
import jax.numpy as jnp
from jax import lax
from jax.experimental import pallas as pl

_INT_MIN = -2147483648


def _skey(q):
    b = lax.bitcast_convert_type(q, jnp.int32)
    return jnp.where(b < 0, jnp.int32(_INT_MIN) - b, b)


def _kth_largest(keys, k, red):
    cnt0 = red(keys >= 0)
    p = jnp.where(cnt0 >= k, jnp.int32(0), jnp.int32(_INT_MIN))
    for beta in range(30, -1, -1):
        c = p + jnp.int32(1 << beta)
        cnt = red(keys >= c)
        p = jnp.where(cnt >= k, c, p)
    return p


def _cumsum_last(x):
    n = x.shape[-1]
    s = x
    sh = 1
    while sh < n:
        z = jnp.zeros(x.shape[:-1] + (sh,), x.dtype)
        s = s + jnp.concatenate([z, s[..., :n - sh]], axis=-1)
        sh *= 2
    return s


def _topk_mask2d(q, k):
    keys = _skey(q)
    red = lambda m: jnp.sum(m.astype(jnp.int32), axis=1, keepdims=True)
    t = _kth_largest(keys, k, red)
    gt = keys > t
    eq = keys == t
    extra = k - red(gt)
    rank = _cumsum_last(eq.astype(jnp.int32))
    return gt | (eq & (rank <= extra))


def _topk_mask3d(q, k):
    keys = _skey(q)

    def red(m):
        return jnp.sum(jnp.sum(m.astype(jnp.int32), axis=2, keepdims=True),
                       axis=1, keepdims=True)

    t = _kth_largest(keys, k, red)
    gt = keys > t
    eq = keys == t
    extra = k - red(gt)
    eqi = eq.astype(jnp.int32)
    within = _cumsum_last(eqi)
    rowtot = jnp.sum(eqi, axis=2)
    exc = _cumsum_last(rowtot) - rowtot
    rank = within + exc[:, :, None]
    return gt | (eq & (rank <= extra))


def _lt_threshold(a, lim):
    t = lim / a
    bits = lax.bitcast_convert_type(t, jnp.int32)
    for _ in range(3):
        t = lax.bitcast_convert_type(bits, jnp.float32)
        bits = jnp.where(a * t >= lim, bits - 1, bits)
    for _ in range(3):
        tn = lax.bitcast_convert_type(bits + 1, jnp.float32)
        bits = jnp.where(a * tn < lim, bits + 1, bits)
    t = lax.bitcast_convert_type(bits + 1, jnp.float32)
    return jnp.where(a == 0.0, jnp.float32(jnp.inf), t)


def _signed_absmax(pmax, nmin):
    return jnp.where(pmax >= -nmin, pmax, nmin)


def _body(x_ref, W1_ref, W2_ref, S1w_ref, S1b_ref, S2w_ref, S2b_ref,
          S3w_ref, S3b_ref, out_ref, sig_ref):
    f32 = jnp.float32
    x = x_ref[...]
    W1 = W1_ref[...]
    W2 = W2_ref[...]

    dot = lambda a, b: lax.dot_general(
        a, b, (((1,), (1,)), ((), ())), preferred_element_type=f32)

    h = jnp.maximum(dot(x, S1w_ref[...]) + S1b_ref[...], 0.0)
    h = jnp.maximum(dot(h, S2w_ref[...]) + S2b_ref[...], 0.0)
    sigma = jnp.tanh(dot(h, S3w_ref[...]) + S3b_ref[...])
    sig_ref[...] = sigma

    sw = jnp.mean(sigma[:, 0:5], axis=1, keepdims=True)
    pw = jnp.mean(sigma[:, 5:10], axis=1, keepdims=True)
    dw = jnp.mean(sigma[:, 10:15], axis=1, keepdims=True)
    tot = jnp.abs(sw) + jnp.abs(pw) + jnp.abs(dw) + 1e-8
    sw, pw, dw = sw / tot, pw / tot, dw / tot

    absW1 = jnp.abs(W1)
    c1 = jnp.sqrt(jnp.sum(W1 * W1, axis=0, keepdims=True))
    mu1 = jnp.mean(W1, axis=0, keepdims=True)
    v1 = jnp.sum((W1 - mu1) ** 2, axis=0, keepdims=True) * (1.0 / 63.0)
    absx = jnp.abs(x)
    T1 = _lt_threshold(absW1, jnp.float32(0.1))
    sp1 = jnp.sum((absx[:, None, :] < T1[None, :, :]).astype(f32),
                  axis=1) * (1.0 / 64.0)
    q1 = sw * (absx * c1) + pw * sp1 + dw * (absx * absx * v1)
    mask1 = _topk_mask2d(q1, 100)

    dt = jnp.tanh(x[:, None, :] * W1[None, :, :])
    absd = jnp.abs(dt)

    absW2 = jnp.abs(W2)
    c2 = jnp.sqrt(jnp.sum(W2 * W2, axis=0, keepdims=True))
    mu2 = jnp.mean(W2, axis=0, keepdims=True)
    v2 = jnp.sum((W2 - mu2) ** 2, axis=0, keepdims=True) * (1.0 / 31.0)
    T2 = _lt_threshold(absW2, jnp.float32(0.1))
    cnt = jnp.zeros_like(dt)
    for j2 in range(32):
        cnt = cnt + (absd < T2[j2:j2 + 1, :, None]).astype(f32)
    q2 = (sw[:, :, None] * (absd * c2[:, :, None])
          + pw[:, :, None] * (cnt * (1.0 / 32.0))
          + dw[:, :, None] * (absd * absd * v2[:, :, None]))
    q2 = jnp.where(mask1[:, None, :], q2, -jnp.inf)
    mask2 = _topk_mask3d(q2, 100)

    dm = jnp.where(mask2, dt, 0.0)
    m1 = jnp.sum(dm, axis=2)
    mean_out = dot(m1, W2) * (1.0 / 100.0)
    pmax = jnp.max(dm, axis=2)
    nmin = jnp.min(dm, axis=2)
    dstar = _signed_absmax(pmax, nmin)
    a = W2[None, :, :] * dstar[:, None, :]
    maxval = _signed_absmax(jnp.max(a, axis=2), jnp.min(a, axis=2))
    alpha = (sigma[:, 15:16] + 1.0) * 0.5
    out_ref[...] = (1.0 - alpha) * mean_out + alpha * maxval


def kernel(x, W1, W2, S1w, S1b, S2w, S2b, S3w, S3b):
    B = x.shape[0]
    Bb = 128
    bcast = lambda shape: pl.BlockSpec(shape, lambda i: (0,) * len(shape))
    out_shape = (jax.ShapeDtypeStruct((B, 32), jnp.float32),
                 jax.ShapeDtypeStruct((B, 16), jnp.float32))
    return pl.pallas_call(
        _body,
        grid=(B // Bb,),
        in_specs=[
            pl.BlockSpec((Bb, 128), lambda i: (i, 0)),
            bcast((64, 128)),
            bcast((32, 64)),
            bcast((64, 128)),
            bcast((1, 64)),
            bcast((32, 64)),
            bcast((1, 32)),
            bcast((16, 32)),
            bcast((1, 16)),
        ],
        out_specs=(pl.BlockSpec((Bb, 32), lambda i: (i, 0)),
                   pl.BlockSpec((Bb, 16), lambda i: (i, 0))),
        out_shape=out_shape,
    )(x, W1, W2, S1w, S1b.reshape(1, 64), S2w, S2b.reshape(1, 32),
      S3w, S3b.reshape(1, 16))


# speedup vs baseline: 5.2515x; 1.3457x over previous
import jax
import jax.numpy as jnp
from jax import lax
from jax.experimental import pallas as pl

_INT_MIN = -2147483648


def _skey(q):
    b = lax.bitcast_convert_type(q, jnp.int32)
    return jnp.where(b < 0, jnp.int32(_INT_MIN) - b, b)


def _kth_largest(keys, k, red):
    cnt0 = red(keys >= 0)
    p = jnp.where(cnt0 >= k, jnp.int32(0), jnp.int32(_INT_MIN))
    for beta in range(30, -1, -1):
        c = p + jnp.int32(1 << beta)
        cnt = red(keys >= c)
        p = jnp.where(cnt >= k, c, p)
    return p


def _cumsum_last(x):
    n = x.shape[-1]
    s = x
    sh = 1
    while sh < n:
        z = jnp.zeros(x.shape[:-1] + (sh,), x.dtype)
        s = s + jnp.concatenate([z, s[..., :n - sh]], axis=-1)
        sh *= 2
    return s


def _topk_mask2d(q, k):
    keys = _skey(q)
    red = lambda m: jnp.sum(m.astype(jnp.int32), axis=1, keepdims=True)
    t = _kth_largest(keys, k, red)
    gt = keys > t
    eq = keys == t
    extra = k - red(gt)
    rank = _cumsum_last(eq.astype(jnp.int32))
    return gt | (eq & (rank <= extra))


def _lane_sums(m, ones_col):
    b, g, n = m.shape
    flat = jnp.reshape(m.astype(jnp.float32), (b * g, n))
    s = lax.dot_general(flat, ones_col, (((1,), (0,)), ((), ())),
                        preferred_element_type=jnp.float32)
    return jnp.reshape(s, (b, g))


def _topk_mask3d(q, k):
    keys = _skey(q)
    b, g, n = q.shape
    ones_col = jnp.ones((n, 1), jnp.float32)

    def red(m):
        s = jnp.sum(_lane_sums(m, ones_col), axis=1, keepdims=True)
        return s.astype(jnp.int32)[:, :, None]

    t = _kth_largest(keys, k, red)
    ge = keys >= t
    eq = keys == t
    eqf = jnp.reshape(eq.astype(jnp.float32), (b * g, n))
    rowtot = jnp.reshape(
        lax.dot_general(eqf, ones_col, (((1,), (0,)), ((), ())),
                        preferred_element_type=jnp.float32), (b, g))
    n_eq = jnp.sum(rowtot, axis=1, keepdims=True)
    n_ge = jnp.sum(_lane_sums(ge, ones_col), axis=1, keepdims=True)
    extra = (jnp.float32(k) - (n_ge - n_eq))[:, :, None]
    tri = (lax.broadcasted_iota(jnp.int32, (n, n), 0)
           <= lax.broadcasted_iota(jnp.int32, (n, n), 1)).astype(jnp.float32)
    within = jnp.reshape(
        lax.dot_general(eqf, tri, (((1,), (0,)), ((), ())),
                        preferred_element_type=jnp.float32), (b, g, n))
    exc = _cumsum_last(rowtot) - rowtot
    rank = within + exc[:, :, None]
    return ge & ((~eq) | (rank <= extra))


def _lt_threshold(a, lim):
    t = lim / a
    bits = lax.bitcast_convert_type(t, jnp.int32)
    for _ in range(3):
        t = lax.bitcast_convert_type(bits, jnp.float32)
        bits = jnp.where(a * t >= lim, bits - 1, bits)
    for _ in range(3):
        tn = lax.bitcast_convert_type(bits + 1, jnp.float32)
        bits = jnp.where(a * tn < lim, bits + 1, bits)
    t = lax.bitcast_convert_type(bits + 1, jnp.float32)
    return jnp.where(a == 0.0, jnp.float32(jnp.inf), t)


def _signed_absmax(pmax, nmin):
    return jnp.where(pmax >= -nmin, pmax, nmin)


def _body(x_ref, W1_ref, W2_ref, S1w_ref, S1b_ref, S2w_ref, S2b_ref,
          S3w_ref, S3b_ref, out_ref, sig_ref):
    f32 = jnp.float32
    x = x_ref[...]
    W1 = W1_ref[...]
    W2 = W2_ref[...]

    dot = lambda a, b: lax.dot_general(
        a, b, (((1,), (1,)), ((), ())), preferred_element_type=f32)

    h = jnp.maximum(dot(x, S1w_ref[...]) + S1b_ref[...], 0.0)
    h = jnp.maximum(dot(h, S2w_ref[...]) + S2b_ref[...], 0.0)
    sigma = jnp.tanh(dot(h, S3w_ref[...]) + S3b_ref[...])
    sig_ref[...] = sigma

    sw = jnp.mean(sigma[:, 0:5], axis=1, keepdims=True)
    pw = jnp.mean(sigma[:, 5:10], axis=1, keepdims=True)
    dw = jnp.mean(sigma[:, 10:15], axis=1, keepdims=True)
    tot = jnp.abs(sw) + jnp.abs(pw) + jnp.abs(dw) + 1e-8
    sw, pw, dw = sw / tot, pw / tot, dw / tot

    absW1 = jnp.abs(W1)
    c1 = jnp.sqrt(jnp.sum(W1 * W1, axis=0, keepdims=True))
    mu1 = jnp.mean(W1, axis=0, keepdims=True)
    v1 = jnp.sum((W1 - mu1) ** 2, axis=0, keepdims=True) * (1.0 / 63.0)
    absx = jnp.abs(x)
    T1 = _lt_threshold(absW1, jnp.float32(0.1))
    sp1 = jnp.sum((absx[:, None, :] < T1[None, :, :]).astype(f32),
                  axis=1) * (1.0 / 64.0)
    q1 = sw * (absx * c1) + pw * sp1 + dw * (absx * absx * v1)
    mask1 = _topk_mask2d(q1, 100)

    dt = jnp.tanh(x[:, None, :] * W1[None, :, :])
    absd = jnp.abs(dt)

    absW2 = jnp.abs(W2)
    c2 = jnp.sqrt(jnp.sum(W2 * W2, axis=0, keepdims=True))
    mu2 = jnp.mean(W2, axis=0, keepdims=True)
    v2 = jnp.sum((W2 - mu2) ** 2, axis=0, keepdims=True) * (1.0 / 31.0)
    T2 = _lt_threshold(absW2, jnp.float32(0.1))
    cnt = jnp.zeros_like(dt)
    for j2 in range(32):
        cnt = cnt + (absd < T2[j2:j2 + 1, :, None]).astype(f32)
    q2 = (sw[:, :, None] * (absd * c2[:, :, None])
          + pw[:, :, None] * (cnt * (1.0 / 32.0))
          + dw[:, :, None] * (absd * absd * v2[:, :, None]))
    q2 = jnp.where(mask1[:, None, :], q2, -jnp.inf)
    mask2 = _topk_mask3d(q2, 100)

    dm = jnp.where(mask2, dt, 0.0)
    Bb = dm.shape[0]
    m1 = jnp.reshape(
        lax.dot_general(jnp.reshape(dm, (Bb * 64, 128)),
                        jnp.ones((128, 1), f32), (((1,), (0,)), ((), ())),
                        preferred_element_type=f32), (Bb, 64))
    mean_out = dot(m1, W2) * (1.0 / 100.0)
    pmax = jnp.max(dm, axis=2)
    nmin = jnp.min(dm, axis=2)
    dstar = _signed_absmax(pmax, nmin)
    a = W2[None, :, :] * dstar[:, None, :]
    maxval = _signed_absmax(jnp.max(a, axis=2), jnp.min(a, axis=2))
    alpha = (sigma[:, 15:16] + 1.0) * 0.5
    out_ref[...] = (1.0 - alpha) * mean_out + alpha * maxval


def kernel(x, W1, W2, S1w, S1b, S2w, S2b, S3w, S3b):
    B = x.shape[0]
    Bb = 128
    bcast = lambda shape: pl.BlockSpec(shape, lambda i: (0,) * len(shape))
    out_shape = (jax.ShapeDtypeStruct((B, 32), jnp.float32),
                 jax.ShapeDtypeStruct((B, 16), jnp.float32))
    return pl.pallas_call(
        _body,
        grid=(B // Bb,),
        in_specs=[
            pl.BlockSpec((Bb, 128), lambda i: (i, 0)),
            bcast((64, 128)),
            bcast((32, 64)),
            bcast((64, 128)),
            bcast((1, 64)),
            bcast((32, 64)),
            bcast((1, 32)),
            bcast((16, 32)),
            bcast((1, 16)),
        ],
        out_specs=(pl.BlockSpec((Bb, 32), lambda i: (i, 0)),
                   pl.BlockSpec((Bb, 16), lambda i: (i, 0))),
        out_shape=out_shape,
    )(x, W1, W2, S1w, S1b.reshape(1, 64), S2w, S2b.reshape(1, 32),
      S3w, S3b.reshape(1, 16))


# speedup vs baseline: 5.6006x; 1.0665x over previous
import jax
import jax.numpy as jnp
from jax import lax
from jax.experimental import pallas as pl

_INT_MIN = -2147483648


def _skey(q):
    b = lax.bitcast_convert_type(q, jnp.int32)
    return jnp.where(b < 0, jnp.int32(_INT_MIN) - b, b)


def _kth_largest(keys, k, red):
    cnt0 = red(keys >= 0)
    p = jnp.where(cnt0 >= k, jnp.int32(0), jnp.int32(_INT_MIN))
    for beta in range(30, -1, -1):
        c = p + jnp.int32(1 << beta)
        cnt = red(keys >= c)
        p = jnp.where(cnt >= k, c, p)
    return p


def _cumsum_last(x):
    n = x.shape[-1]
    s = x
    sh = 1
    while sh < n:
        z = jnp.zeros(x.shape[:-1] + (sh,), x.dtype)
        s = s + jnp.concatenate([z, s[..., :n - sh]], axis=-1)
        sh *= 2
    return s


def _topk_mask2d(q, k):
    keys = _skey(q)
    red = lambda m: jnp.sum(m.astype(jnp.int32), axis=1, keepdims=True)
    t = _kth_largest(keys, k, red)
    gt = keys > t
    eq = keys == t
    extra = k - red(gt)
    rank = _cumsum_last(eq.astype(jnp.int32))
    return gt | (eq & (rank <= extra))


def _lane_sums(m, ones_col):
    b, g, n = m.shape
    flat = jnp.reshape(m.astype(jnp.float32), (b * g, n))
    s = lax.dot_general(flat, ones_col, (((1,), (0,)), ((), ())),
                        preferred_element_type=jnp.float32)
    return jnp.reshape(s, (b, g))


def _topk_mask3d(q, k):
    keys = _skey(q)
    b, g, n = q.shape
    ones_col = jnp.ones((n, 1), jnp.float32)

    def red(m):
        s = jnp.sum(_lane_sums(m, ones_col), axis=1, keepdims=True)
        return s.astype(jnp.int32)[:, :, None]

    t = _kth_largest(keys, k, red)
    ge = keys >= t
    eq = keys == t
    eqf = jnp.reshape(eq.astype(jnp.float32), (b * g, n))
    rowtot = jnp.reshape(
        lax.dot_general(eqf, ones_col, (((1,), (0,)), ((), ())),
                        preferred_element_type=jnp.float32), (b, g))
    n_eq = jnp.sum(rowtot, axis=1, keepdims=True)
    n_ge = jnp.sum(_lane_sums(ge, ones_col), axis=1, keepdims=True)
    extra = (jnp.float32(k) - (n_ge - n_eq))[:, :, None]
    tri = (lax.broadcasted_iota(jnp.int32, (n, n), 0)
           <= lax.broadcasted_iota(jnp.int32, (n, n), 1)).astype(jnp.float32)
    within = jnp.reshape(
        lax.dot_general(eqf, tri, (((1,), (0,)), ((), ())),
                        preferred_element_type=jnp.float32), (b, g, n))
    exc = _cumsum_last(rowtot) - rowtot
    rank = within + exc[:, :, None]
    return ge & ((~eq) | (rank <= extra))


def _bitonic_axis0(v):
    import numpy as np
    n = v.shape[0]
    k = 2
    while k <= n:
        j = k // 2
        while j >= 1:
            nb = n // (2 * j)
            a = jnp.reshape(v, (nb, 2, j) + v.shape[1:])
            lo, hi = a[:, 0], a[:, 1]
            mn = jnp.minimum(lo, hi)
            mx = jnp.maximum(lo, hi)
            bidx = lax.broadcasted_iota(jnp.int32,
                                        (nb,) + (1,) * (v.ndim), 0)
            ascb = ((bidx * (2 * j)) & k) == 0
            first = jnp.where(ascb, mn, mx)
            second = jnp.where(ascb, mx, mn)
            v = jnp.reshape(jnp.stack([first, second], axis=1),
                            (n,) + v.shape[1:])
            j //= 2
        k *= 2
    return v


def _count_le_tree(u, Ts):
    pv = lambda idx: Ts[idx][None, :, None]
    f32 = jnp.float32
    m4 = u >= pv(15)
    p = jnp.where(m4, pv(23), pv(7))
    m3 = u >= p
    p = jnp.where(m4, jnp.where(m3, pv(27), pv(19)),
                  jnp.where(m3, pv(11), pv(3)))
    m2 = u >= p
    sel8 = [pv(1 + 4 * t) for t in range(8)]
    p = jnp.where(m4,
                  jnp.where(m3, jnp.where(m2, sel8[7], sel8[6]),
                            jnp.where(m2, sel8[5], sel8[4])),
                  jnp.where(m3, jnp.where(m2, sel8[3], sel8[2]),
                            jnp.where(m2, sel8[1], sel8[0])))
    m1 = u >= p
    sel16 = [pv(2 * t) for t in range(16)]

    def pick(ms, lo, hi):
        if hi - lo == 1:
            return sel16[lo]
        mid = (lo + hi) // 2
        return jnp.where(ms[0], pick(ms[1:], mid, hi), pick(ms[1:], lo, mid))

    p = pick([m4, m3, m2, m1], 0, 16)
    m0 = u >= p
    m5 = u >= pv(31)
    ub = (jnp.where(m4, f32(16.0), f32(0.0))
          + jnp.where(m3, f32(8.0), f32(0.0))
          + jnp.where(m2, f32(4.0), f32(0.0))
          + jnp.where(m1, f32(2.0), f32(0.0))
          + jnp.where(m0, f32(1.0), f32(0.0))
          + jnp.where(m5, f32(1.0), f32(0.0)))
    return ub


def _lt_threshold(a, lim):
    t = lim / a
    bits = lax.bitcast_convert_type(t, jnp.int32)
    for _ in range(3):
        t = lax.bitcast_convert_type(bits, jnp.float32)
        bits = jnp.where(a * t >= lim, bits - 1, bits)
    for _ in range(3):
        tn = lax.bitcast_convert_type(bits + 1, jnp.float32)
        bits = jnp.where(a * tn < lim, bits + 1, bits)
    t = lax.bitcast_convert_type(bits + 1, jnp.float32)
    return jnp.where(a == 0.0, jnp.float32(jnp.inf), t)


def _signed_absmax(pmax, nmin):
    return jnp.where(pmax >= -nmin, pmax, nmin)


def _body(x_ref, W1_ref, W2_ref, S1w_ref, S1b_ref, S2w_ref, S2b_ref,
          S3w_ref, S3b_ref, out_ref, sig_ref):
    f32 = jnp.float32
    x = x_ref[...]
    W1 = W1_ref[...]
    W2 = W2_ref[...]

    dot = lambda a, b: lax.dot_general(
        a, b, (((1,), (1,)), ((), ())), preferred_element_type=f32)

    h = jnp.maximum(dot(x, S1w_ref[...]) + S1b_ref[...], 0.0)
    h = jnp.maximum(dot(h, S2w_ref[...]) + S2b_ref[...], 0.0)
    sigma = jnp.tanh(dot(h, S3w_ref[...]) + S3b_ref[...])
    sig_ref[...] = sigma

    sw = jnp.mean(sigma[:, 0:5], axis=1, keepdims=True)
    pw = jnp.mean(sigma[:, 5:10], axis=1, keepdims=True)
    dw = jnp.mean(sigma[:, 10:15], axis=1, keepdims=True)
    tot = jnp.abs(sw) + jnp.abs(pw) + jnp.abs(dw) + 1e-8
    sw, pw, dw = sw / tot, pw / tot, dw / tot

    absW1 = jnp.abs(W1)
    c1 = jnp.sqrt(jnp.sum(W1 * W1, axis=0, keepdims=True))
    mu1 = jnp.mean(W1, axis=0, keepdims=True)
    v1 = jnp.sum((W1 - mu1) ** 2, axis=0, keepdims=True) * (1.0 / 63.0)
    absx = jnp.abs(x)
    T1 = _lt_threshold(absW1, jnp.float32(0.1))
    sp1 = jnp.sum((absx[:, None, :] < T1[None, :, :]).astype(f32),
                  axis=1) * (1.0 / 64.0)
    q1 = sw * (absx * c1) + pw * sp1 + dw * (absx * absx * v1)

    dt = jnp.tanh(x[:, None, :] * W1[None, :, :])
    absd = jnp.abs(dt)

    absW2 = jnp.abs(W2)
    c2 = jnp.sqrt(jnp.sum(W2 * W2, axis=0, keepdims=True))
    mu2 = jnp.mean(W2, axis=0, keepdims=True)
    v2 = jnp.sum((W2 - mu2) ** 2, axis=0, keepdims=True) * (1.0 / 31.0)
    T2 = _lt_threshold(absW2, jnp.float32(0.1))
    T2s = _bitonic_axis0(T2)
    ub = _count_le_tree(absd, T2s)
    cnt = jnp.float32(32.0) - ub
    q2 = (sw[:, :, None] * (absd * c2[:, :, None])
          + pw[:, :, None] * (cnt * (1.0 / 32.0))
          + dw[:, :, None] * (absd * absd * v2[:, :, None]))
    mask1 = _topk_mask2d(q1, 100)
    q2 = jnp.where(mask1[:, None, :], q2, -jnp.inf)
    mask2 = _topk_mask3d(q2, 100)

    dm = jnp.where(mask2, dt, 0.0)
    Bb = dm.shape[0]
    m1 = jnp.reshape(
        lax.dot_general(jnp.reshape(dm, (Bb * 64, 128)),
                        jnp.ones((128, 1), f32), (((1,), (0,)), ((), ())),
                        preferred_element_type=f32), (Bb, 64))
    mean_out = dot(m1, W2) * (1.0 / 100.0)
    pmax = jnp.max(dm, axis=2)
    nmin = jnp.min(dm, axis=2)
    dstar = _signed_absmax(pmax, nmin)
    a = W2[None, :, :] * dstar[:, None, :]
    maxval = _signed_absmax(jnp.max(a, axis=2), jnp.min(a, axis=2))
    alpha = (sigma[:, 15:16] + 1.0) * 0.5
    out_ref[...] = (1.0 - alpha) * mean_out + alpha * maxval


def kernel(x, W1, W2, S1w, S1b, S2w, S2b, S3w, S3b):
    B = x.shape[0]
    Bb = 128
    bcast = lambda shape: pl.BlockSpec(shape, lambda i: (0,) * len(shape))
    out_shape = (jax.ShapeDtypeStruct((B, 32), jnp.float32),
                 jax.ShapeDtypeStruct((B, 16), jnp.float32))
    return pl.pallas_call(
        _body,
        grid=(B // Bb,),
        in_specs=[
            pl.BlockSpec((Bb, 128), lambda i: (i, 0)),
            bcast((64, 128)),
            bcast((32, 64)),
            bcast((64, 128)),
            bcast((1, 64)),
            bcast((32, 64)),
            bcast((1, 32)),
            bcast((16, 32)),
            bcast((1, 16)),
        ],
        out_specs=(pl.BlockSpec((Bb, 32), lambda i: (i, 0)),
                   pl.BlockSpec((Bb, 16), lambda i: (i, 0))),
        out_shape=out_shape,
    )(x, W1, W2, S1w, S1b.reshape(1, 64), S2w, S2b.reshape(1, 32),
      S3w, S3b.reshape(1, 16))
